# baseline (speedup 1.0000x reference)
import jax
import jax.numpy as jnp
from jax import lax
from jax.experimental import pallas as pl
from jax.experimental.pallas import tpu as pltpu
from jax.experimental.pallas import tpu_sc as plsc

B = 16
CLS = 100
POOL = 100
CTX_LEN = 12
D = 512
SEQ = 77
SUF = SEQ - 1 - CTX_LEN * 2
NC_SUF = SEQ - 1 - CTX_LEN
NC_SEQ = 1 + 2 * CTX_LEN + NC_SUF

CB = 50
NCB = CLS // CB


def _sc_gather_body(ig, ia, gp, ap, out, idx_v, rows_v, sem):
    c = lax.axis_index("c")
    s = lax.axis_index("s")
    wid = s * 2 + c

    @pl.when(wid == 0)
    def _():
        pltpu.sync_copy(ig, idx_v)
        pltpu.async_copy(gp.at[idx_v], rows_v, sem).wait()
        for i in range(B):
            pltpu.sync_copy(rows_v.at[i], out.at[i // 2, i % 2])

    @pl.when(wid == 1)
    def _():
        pltpu.sync_copy(ia, idx_v)
        pltpu.async_copy(ap.at[idx_v], rows_v, sem).wait()
        for i in range(B):
            pltpu.sync_copy(rows_v.at[i], out.at[8 + i // 2, i % 2])


def _sc_gather(ig, ia, gp, ap):
    return pl.kernel(
        _sc_gather_body,
        out_type=jax.ShapeDtypeStruct((B, 2, CTX_LEN, D), jnp.float32),
        mesh=plsc.VectorSubcoreMesh(core_axis_name="c", subcore_axis_name="s"),
        compiler_params=pltpu.CompilerParams(use_tc_tiling_on_sc=False),
        scratch_types=[
            pltpu.VMEM((B,), jnp.int32),
            pltpu.VMEM((B, CTX_LEN, D), jnp.float32),
            pltpu.SemaphoreType.DMA,
        ],
    )(ig, ia, gp, ap)


def _prompts_body(ctx, pre, suf, out):
    s = pl.program_id(0)
    cb = s % NCB
    out[:, 0:1, :] = pre[...]
    out[:, 1:1 + CTX_LEN, :] = jnp.broadcast_to(
        ctx[0, 0], (CB, CTX_LEN, D))
    out[:, 1 + CTX_LEN:1 + 2 * CTX_LEN, :] = jnp.broadcast_to(
        ctx[0, 1], (CB, CTX_LEN, D))
    out[:, 1 + 2 * CTX_LEN:SEQ, :] = suf[pl.ds(cb * CB, CB)]


def _build_prompts_call():
    return pl.pallas_call(
        _prompts_body,
        grid=(B * NCB,),
        in_specs=[
            pl.BlockSpec((1, 2, CTX_LEN, D), lambda s: (s // NCB, 0, 0, 0)),
            pl.BlockSpec((CB, 1, D), lambda s: (s % NCB, 0, 0)),
            pl.BlockSpec((CLS, SUF, D), lambda s: (0, 0, 0)),
        ],
        out_specs=pl.BlockSpec((CB, SEQ, D), lambda s: (s, 0, 0)),
        out_shape=jax.ShapeDtypeStruct((B * CLS, SEQ, D), jnp.float32),
    )


def _nc_body(ncp, gp, ap, ncs, nctok, tokp, out, nc_tok_out, tok_out):
    out[:, 0:1, :] = jnp.broadcast_to(ncp[...], (CB, 1, D))
    out[:, 1:1 + CTX_LEN, :] = gp[...]
    out[:, 1 + CTX_LEN:1 + 2 * CTX_LEN, :] = ap[...]
    out[:, 1 + 2 * CTX_LEN:NC_SEQ, :] = jnp.broadcast_to(
        ncs[...], (CB, NC_SUF, D))
    nc_tok_out[...] = jnp.broadcast_to(nctok[...], (POOL, SEQ))
    t = tokp[...]
    for b in range(B):
        tok_out[pl.ds(b * CLS, CLS), :] = t


def _build_nc_call():
    return pl.pallas_call(
        _nc_body,
        grid=(NCB,),
        in_specs=[
            pl.BlockSpec((1, 1, D), lambda i: (0, 0, 0)),
            pl.BlockSpec((CB, CTX_LEN, D), lambda i: (i, 0, 0)),
            pl.BlockSpec((CB, CTX_LEN, D), lambda i: (i, 0, 0)),
            pl.BlockSpec((1, NC_SUF, D), lambda i: (0, 0, 0)),
            pl.BlockSpec((1, SEQ), lambda i: (0, 0)),
            pl.BlockSpec((CLS, SEQ), lambda i: (0, 0)),
        ],
        out_specs=[
            pl.BlockSpec((CB, NC_SEQ, D), lambda i: (i, 0, 0)),
            pl.BlockSpec((POOL, SEQ), lambda i: (0, 0)),
            pl.BlockSpec((B * CLS, SEQ), lambda i: (0, 0)),
        ],
        out_shape=[
            jax.ShapeDtypeStruct((POOL, NC_SEQ, D), jnp.float32),
            jax.ShapeDtypeStruct((POOL, SEQ), jnp.int32),
            jax.ShapeDtypeStruct((B * CLS, SEQ), jnp.int32),
        ],
    )


def kernel(indices_g, indices_a, global_prompt, attribute_prompt,
           token_prefix, token_suffix, nc_token_prefix, nc_token_suffix,
           tokenized_prompts, nc_tokenized_prompts):
    ig = indices_g.astype(jnp.int32)
    ia = indices_a.astype(jnp.int32)
    tokp = tokenized_prompts.astype(jnp.int32)
    nctok = nc_tokenized_prompts.astype(jnp.int32)

    ctx = _sc_gather(ig, ia, global_prompt, attribute_prompt)

    prompts = _build_prompts_call()(ctx, token_prefix, token_suffix)

    nc_prompts, nc_tok, tok = _build_nc_call()(
        nc_token_prefix, global_prompt, attribute_prompt,
        nc_token_suffix, nctok, tokp)

    return (prompts, tok, nc_prompts, nc_tok)


# speedup vs baseline: 1.0045x; 1.0045x over previous
import jax
import jax.numpy as jnp
from jax import lax
from jax.experimental import pallas as pl
from jax.experimental.pallas import tpu as pltpu
from jax.experimental.pallas import tpu_sc as plsc

B = 16
CLS = 100
POOL = 100
CTX_LEN = 12
D = 512
SEQ = 77
SUF = SEQ - 1 - CTX_LEN * 2
NC_SUF = SEQ - 1 - CTX_LEN
NC_SEQ = 1 + 2 * CTX_LEN + NC_SUF

CB = 50
NCB = CLS // CB


def _sc_gather_body(ig, ia, gp, ap, out, idx_v, rows_v, sem):
    c = lax.axis_index("c")
    s = lax.axis_index("s")
    wid = s * 2 + c

    @pl.when(wid == 0)
    def _():
        pltpu.sync_copy(ig, idx_v)
        pltpu.async_copy(gp.at[idx_v], rows_v, sem).wait()
        cps = [pltpu.async_copy(rows_v.at[i], out.at[i // 2, i % 2], sem)
               for i in range(B)]
        for cp in cps:
            cp.wait()

    @pl.when(wid == 1)
    def _():
        pltpu.sync_copy(ia, idx_v)
        pltpu.async_copy(ap.at[idx_v], rows_v, sem).wait()
        cps = [pltpu.async_copy(rows_v.at[i], out.at[8 + i // 2, i % 2], sem)
               for i in range(B)]
        for cp in cps:
            cp.wait()


def _sc_gather(ig, ia, gp, ap):
    return pl.kernel(
        _sc_gather_body,
        out_type=jax.ShapeDtypeStruct((B, 2, CTX_LEN, D), jnp.float32),
        mesh=plsc.VectorSubcoreMesh(core_axis_name="c", subcore_axis_name="s"),
        compiler_params=pltpu.CompilerParams(use_tc_tiling_on_sc=False),
        scratch_types=[
            pltpu.VMEM((B,), jnp.int32),
            pltpu.VMEM((B, CTX_LEN, D), jnp.float32),
            pltpu.SemaphoreType.DMA,
        ],
    )(ig, ia, gp, ap)


def _prompts_body(ctx, pre, suf, ncp, gp, ap, ncs, nctok, tokp,
                  out, nc_out, nc_tok_out, tok_out):
    s = pl.program_id(0)
    cb = s % NCB
    out[:, 0:1, :] = pre[...]
    out[:, 1:1 + CTX_LEN, :] = jnp.broadcast_to(
        ctx[0, 0], (CB, CTX_LEN, D))
    out[:, 1 + CTX_LEN:1 + 2 * CTX_LEN, :] = jnp.broadcast_to(
        ctx[0, 1], (CB, CTX_LEN, D))
    out[:, 1 + 2 * CTX_LEN:SEQ, :] = suf[pl.ds(cb * CB, CB)]

    @pl.when(s == 0)
    def _():
        nc_out[:, 0:1, :] = jnp.broadcast_to(ncp[...], (POOL, 1, D))
        nc_out[:, 1:1 + CTX_LEN, :] = gp[...]
        nc_out[:, 1 + CTX_LEN:1 + 2 * CTX_LEN, :] = ap[...]
        nc_out[:, 1 + 2 * CTX_LEN:NC_SEQ, :] = jnp.broadcast_to(
            ncs[...], (POOL, NC_SUF, D))
        nc_tok_out[...] = jnp.broadcast_to(nctok[...], (POOL, SEQ))
        t = tokp[...]
        for b in range(B):
            tok_out[pl.ds(b * CLS, CLS), :] = t


def _build_prompts_call():
    cst3 = lambda s: (0, 0, 0)
    cst2 = lambda s: (0, 0)
    return pl.pallas_call(
        _prompts_body,
        grid=(B * NCB,),
        in_specs=[
            pl.BlockSpec((1, 2, CTX_LEN, D), lambda s: (s // NCB, 0, 0, 0)),
            pl.BlockSpec((CB, 1, D), lambda s: (s % NCB, 0, 0)),
            pl.BlockSpec((CLS, SUF, D), cst3),
            pl.BlockSpec((1, 1, D), cst3),
            pl.BlockSpec((POOL, CTX_LEN, D), cst3),
            pl.BlockSpec((POOL, CTX_LEN, D), cst3),
            pl.BlockSpec((1, NC_SUF, D), cst3),
            pl.BlockSpec((1, SEQ), cst2),
            pl.BlockSpec((CLS, SEQ), cst2),
        ],
        out_specs=[
            pl.BlockSpec((CB, SEQ, D), lambda s: (s, 0, 0)),
            pl.BlockSpec((POOL, NC_SEQ, D), cst3),
            pl.BlockSpec((POOL, SEQ), cst2),
            pl.BlockSpec((B * CLS, SEQ), cst2),
        ],
        out_shape=[
            jax.ShapeDtypeStruct((B * CLS, SEQ, D), jnp.float32),
            jax.ShapeDtypeStruct((POOL, NC_SEQ, D), jnp.float32),
            jax.ShapeDtypeStruct((POOL, SEQ), jnp.int32),
            jax.ShapeDtypeStruct((B * CLS, SEQ), jnp.int32),
        ],
    )


def kernel(indices_g, indices_a, global_prompt, attribute_prompt,
           token_prefix, token_suffix, nc_token_prefix, nc_token_suffix,
           tokenized_prompts, nc_tokenized_prompts):
    ig = indices_g.astype(jnp.int32)
    ia = indices_a.astype(jnp.int32)
    tokp = tokenized_prompts.astype(jnp.int32)
    nctok = nc_tokenized_prompts.astype(jnp.int32)

    ctx = _sc_gather(ig, ia, global_prompt, attribute_prompt)

    prompts, nc_prompts, nc_tok, tok = _build_prompts_call()(
        ctx, token_prefix, token_suffix, nc_token_prefix, global_prompt,
        attribute_prompt, nc_token_suffix, nctok, tokp)

    return (prompts, tok, nc_prompts, nc_tok)
